# Initial kernel scaffold; baseline (speedup 1.0000x reference)
import jax
import jax.numpy as jnp
from jax.experimental import pallas as pl


def kernel(edge_index, conv64, conv128, conv256, Wi0_1, Wi1_1, bi_1, Wh0_1, Wh1_1, bh_1, Wo0_1, Wo1_1, bo_1, Wi0_2, Wi1_2, bi_2, Wh0_2, Wh1_2, bh_2, Wo0_2, Wo1_2, bo_2, Wi0_3, Wi1_3, bi_3, Wh0_3, Wh1_3, bh_3, Wo0_3, Wo1_3, bo_3):
    raise NotImplementedError("write your pallas kernel here")


# speedup vs baseline: 1.5574x; 1.5574x over previous
import functools

import jax
import jax.numpy as jnp
from jax import lax
from jax.experimental import pallas as pl
from jax.experimental.pallas import tpu as pltpu
from jax.experimental.pallas import tpu_sc as plsc

N = 10000
E = 160000
E2 = 2 * E
HID = 256
HL = 12
WC = 128

NSUB = 16
NCORE = 2

RANGE = 5056
ACC_ROWS = 5120
ZERO_ROWS_PER_TILE = ACC_ROWS // NSUB
OUT_ROWS = 2 * ACC_ROWS

K_COL = -(-(E2 // NSUB) // 128)
K_EDGE = -(-(E2 // (NSUB * NCORE)) // 128)

_f32 = jnp.float32


@functools.lru_cache(maxsize=None)
def _spmm_colsplit():
    mesh = plsc.VectorSubcoreMesh(core_axis_name="c", subcore_axis_name="s")
    out_t = [jax.ShapeDtypeStruct((OUT_ROWS, WC), _f32)] * 2

    @functools.partial(
        pl.kernel,
        out_type=out_t,
        mesh=mesh,
        scratch_types=[
            pltpu.VMEM((K_COL, 128), jnp.int32),
            pltpu.VMEM((K_COL, 128), jnp.int32),
            pltpu.VMEM((K_COL, 128), jnp.int32),
            pltpu.VMEM((128, WC), _f32),
            pltpu.VMEM_SHARED((ACC_ROWS, WC), _f32),
            pltpu.SemaphoreType.DMA,
        ],
    )
    def k(xa, xb, gidx, sidx, zeros, outa, outb,
          gbuf, sbuf0, sbuf1, rows, acc, sem):
        c = lax.axis_index("c")
        s = lax.axis_index("s")
        pltpu.sync_copy(gidx.at[s], gbuf)
        pltpu.sync_copy(sidx.at[0, s], sbuf0)
        pltpu.sync_copy(sidx.at[1, s], sbuf1)
        z0 = s * ZERO_ROWS_PER_TILE

        def body(x_ref, out_ref):
            for p, sbuf in ((0, sbuf0), (1, sbuf1)):
                pltpu.sync_copy(zeros,
                                acc.at[pl.ds(z0, ZERO_ROWS_PER_TILE)])
                plsc.subcore_barrier()

                def step(i, carry):
                    pltpu.async_copy(x_ref.at[gbuf.at[i]], rows, sem).wait()
                    pltpu.sync_copy(rows, acc.at[sbuf.at[i]], add=True)
                    return carry
                lax.fori_loop(0, K_COL, step, 0)
                plsc.subcore_barrier()
                pltpu.sync_copy(
                    acc.at[pl.ds(z0, ZERO_ROWS_PER_TILE)],
                    out_ref.at[pl.ds(p * ACC_ROWS + z0, ZERO_ROWS_PER_TILE)])
                plsc.subcore_barrier()

        @pl.when(c == 0)
        def _():
            body(xa, outa)

        @pl.when(c == 1)
        def _():
            body(xb, outb)

    return k


@functools.lru_cache(maxsize=None)
def _spmm_edgesplit():
    mesh = plsc.VectorSubcoreMesh(core_axis_name="c", subcore_axis_name="s")

    @functools.partial(
        pl.kernel,
        out_type=jax.ShapeDtypeStruct((NCORE, OUT_ROWS, WC), _f32),
        mesh=mesh,
        scratch_types=[
            pltpu.VMEM((K_EDGE, 128), jnp.int32),
            pltpu.VMEM((K_EDGE, 128), jnp.int32),
            pltpu.VMEM((K_EDGE, 128), jnp.int32),
            pltpu.VMEM((128, WC), _f32),
            pltpu.VMEM_SHARED((ACC_ROWS, WC), _f32),
            pltpu.SemaphoreType.DMA,
        ],
    )
    def k(x, gidx, sidx, zeros, out, gbuf, sbuf0, sbuf1, rows, acc, sem):
        c = lax.axis_index("c")
        s = lax.axis_index("s")
        w = s * NCORE + c
        pltpu.sync_copy(gidx.at[w], gbuf)
        pltpu.sync_copy(sidx.at[0, w], sbuf0)
        pltpu.sync_copy(sidx.at[1, w], sbuf1)
        z0 = s * ZERO_ROWS_PER_TILE

        for p, sbuf in ((0, sbuf0), (1, sbuf1)):
            pltpu.sync_copy(zeros, acc.at[pl.ds(z0, ZERO_ROWS_PER_TILE)])
            plsc.subcore_barrier()

            def step(i, carry):
                pltpu.async_copy(x.at[gbuf.at[i]], rows, sem).wait()
                pltpu.sync_copy(rows, acc.at[sbuf.at[i]], add=True)
                return carry
            lax.fori_loop(0, K_EDGE, step, 0)
            plsc.subcore_barrier()
            pltpu.sync_copy(
                acc.at[pl.ds(z0, ZERO_ROWS_PER_TILE)],
                out.at[c, pl.ds(p * ACC_ROWS + z0, ZERO_ROWS_PER_TILE)])
            plsc.subcore_barrier()

    return k


R = 1000
GRID = N // R

_dot = functools.partial(jnp.dot, preferred_element_type=_f32)


@functools.lru_cache(maxsize=None)
def _in_proj(d):
    def body(f_ref, w0_ref, w1_ref, b_ref, t0a, t0b, t1a, t1b):
        f = f_ref[...]
        t0 = _dot(f, w0_ref[...]) + b_ref[...]
        t1 = _dot(f, w1_ref[...])
        t0a[...] = t0[:, :128]
        t0b[...] = t0[:, 128:]
        t1a[...] = t1[:, :128]
        t1b[...] = t1[:, 128:]

    half = pl.BlockSpec((R, 128), lambda i: (i, 0))
    return pl.pallas_call(
        body,
        grid=(GRID,),
        in_specs=[
            pl.BlockSpec((R, d), lambda i: (i, 0)),
            pl.BlockSpec((d, HID), lambda i: (0, 0)),
            pl.BlockSpec((d, HID), lambda i: (0, 0)),
            pl.BlockSpec((1, HID), lambda i: (0, 0)),
        ],
        out_specs=[half, half, half, half],
        out_shape=[jax.ShapeDtypeStruct((N, 128), _f32)] * 4,
    )


@functools.lru_cache(maxsize=None)
def _add_relu():
    def body(t0a, t0b, aa, ab, xa, xb):
        xa[...] = jnp.maximum(t0a[...] + aa[...], 0.0)
        xb[...] = jnp.maximum(t0b[...] + ab[...], 0.0)

    half = pl.BlockSpec((R, 128), lambda i: (i, 0))
    return pl.pallas_call(
        body,
        grid=(GRID,),
        in_specs=[half, half, half, half],
        out_specs=[half, half],
        out_shape=[jax.ShapeDtypeStruct((N, 128), _f32)] * 2,
    )


@functools.lru_cache(maxsize=None)
def _gc_hidden():
    def body(xa_ref, xb_ref, aa_ref, ab_ref, w0_ref, w1_ref, b_ref,
             ya_ref, yb_ref):
        t = _dot(xa_ref[...], w0_ref[0:128, :])
        t += _dot(xb_ref[...], w0_ref[128:256, :])
        t += _dot(aa_ref[...], w1_ref[0:128, :])
        t += _dot(ab_ref[...], w1_ref[128:256, :])
        y = jnp.maximum(t + b_ref[...], 0.0)
        ya_ref[...] = y[:, :128]
        yb_ref[...] = y[:, 128:]

    half = pl.BlockSpec((R, 128), lambda i: (i, 0))
    wfull = pl.BlockSpec((HID, HID), lambda i: (0, 0))
    return pl.pallas_call(
        body,
        grid=(GRID,),
        in_specs=[half, half, half, half, wfull, wfull,
                  pl.BlockSpec((1, HID), lambda i: (0, 0))],
        out_specs=[half, half],
        out_shape=[jax.ShapeDtypeStruct((N, 128), _f32)] * 2,
    )


@functools.lru_cache(maxsize=None)
def _out_proj():
    def body(xa_ref, xb_ref, ra_ref, rb_ref, w_ref, h_ref):
        a = xa_ref[...] + ra_ref[...]
        b = xb_ref[...] + rb_ref[...]
        h_ref[...] = _dot(a, w_ref[0:128, :]) + _dot(b, w_ref[128:256, :])

    half = pl.BlockSpec((R, 128), lambda i: (i, 0))
    return pl.pallas_call(
        body,
        grid=(GRID,),
        in_specs=[half, half, half, half,
                  pl.BlockSpec((HID, 128), lambda i: (0, 0))],
        out_specs=pl.BlockSpec((R, 128), lambda i: (i, 0)),
        out_shape=jax.ShapeDtypeStruct((N, 128), _f32),
    )


@functools.lru_cache(maxsize=None)
def _out_final():
    def body(xa_ref, xb_ref, ra_ref, rb_ref, p0_ref, p1_ref, w_ref, b_ref,
             v_ref):
        a = xa_ref[...] + ra_ref[...]
        b = xb_ref[...] + rb_ref[...]
        t = _dot(a, w_ref[0:128, :]) + _dot(b, w_ref[128:256, :])
        t += p0_ref[...] + p1_ref[...] + b_ref[...]
        v_ref[...] = jnp.maximum(t, 0.0)

    half = pl.BlockSpec((R, 128), lambda i: (i, 0))
    return pl.pallas_call(
        body,
        grid=(GRID,),
        in_specs=[half, half, half, half, half, half,
                  pl.BlockSpec((HID, 128), lambda i: (0, 0)),
                  pl.BlockSpec((1, 128), lambda i: (0, 0))],
        out_specs=pl.BlockSpec((R, 128), lambda i: (i, 0)),
        out_shape=jax.ShapeDtypeStruct((N, 128), _f32),
    )


def _pad_to(x, n, fill):
    return jnp.pad(x, (0, n - x.shape[0]), constant_values=fill)


def _stage(feat, d, idx_col, idx_edge, Wi0, Wi1, bi, Wh0, Wh1, bh,
           Wo0, Wo1, bo):
    gcol, scol = idx_col
    gedge, sedge = idx_edge
    zeros = jnp.zeros((ZERO_ROWS_PER_TILE, WC), _f32)

    def unstack(o):
        return jnp.concatenate([o[:RANGE], o[ACC_ROWS:ACC_ROWS + N - RANGE]])

    t0a, t0b, t1a, t1b = _in_proj(d)(feat, Wi0, Wi1, bi.reshape(1, HID))
    aa, ab = _spmm_colsplit()(t1a, t1b, gcol, scol, zeros)
    xa, xb = _add_relu()(t0a, t0b, unstack(aa), unstack(ab))
    ra, rb = xa, xb

    for i in range(HL):
        aa, ab = _spmm_colsplit()(xa, xb, gcol, scol, zeros)
        xa, xb = _gc_hidden()(xa, xb, unstack(aa), unstack(ab),
                              Wh0[i], Wh1[i], bh[i].reshape(1, HID))

    Wo1p = jnp.pad(Wo1, ((0, 0), (0, 128 - Wo1.shape[1])))
    Wo0p = jnp.pad(Wo0, ((0, 0), (0, 128 - Wo0.shape[1])))
    bop = jnp.pad(bo, (0, 128 - bo.shape[0])).reshape(1, 128)
    h = _out_proj()(xa, xb, ra, rb, Wo1p)
    p = _spmm_edgesplit()(h, gedge, sedge, zeros)
    v = _out_final()(xa, xb, ra, rb, unstack(p[0]), unstack(p[1]), Wo0p, bop)
    return v[:, :3]


def kernel(edge_index, conv64, conv128, conv256,
           Wi0_1, Wi1_1, bi_1, Wh0_1, Wh1_1, bh_1, Wo0_1, Wo1_1, bo_1,
           Wi0_2, Wi1_2, bi_2, Wh0_2, Wh1_2, bh_2, Wo0_2, Wo1_2, bo_2,
           Wi0_3, Wi1_3, bi_3, Wh0_3, Wh1_3, bh_3, Wo0_3, Wo1_3, bo_3):
    src = edge_index[0]
    dst = edge_index[1]
    gat = jnp.concatenate([src, dst])
    sct = jnp.concatenate([dst, src])

    def passes(sct_pad):
        out = []
        for p in (0, 1):
            lo = p * RANGE
            loc = sct_pad - lo
            out.append(jnp.where((loc >= 0) & (loc < RANGE), loc, RANGE))
        return jnp.stack(out)

    ncol = NSUB * K_COL * 128
    gcol = _pad_to(gat, ncol, 0).reshape(NSUB, K_COL, 128)
    scol = passes(_pad_to(sct, ncol, N)).reshape(2, NSUB, K_COL, 128)

    nedge = NSUB * NCORE * K_EDGE * 128
    gedge = _pad_to(gat, nedge, 0).reshape(NSUB * NCORE, K_EDGE, 128)
    sedge = passes(_pad_to(sct, nedge, N)).reshape(2, NSUB * NCORE, K_EDGE, 128)

    idx_col = (gcol, scol)
    idx_edge = (gedge, sedge)

    v1 = _stage(conv64, 64, idx_col, idx_edge,
                Wi0_1, Wi1_1, bi_1, Wh0_1, Wh1_1, bh_1, Wo0_1, Wo1_1, bo_1)
    v2 = _stage(conv128, 128, idx_col, idx_edge,
                Wi0_2, Wi1_2, bi_2, Wh0_2, Wh1_2, bh_2, Wo0_2, Wo1_2, bo_2)
    v3 = _stage(conv256, 256, idx_col, idx_edge,
                Wi0_3, Wi1_3, bi_3, Wh0_3, Wh1_3, bh_3, Wo0_3, Wo1_3, bo_3)
    return jnp.stack([v1, v2, v3], axis=0)
